# baseline (speedup 1.0000x reference)
import functools

import jax
import jax.numpy as jnp
from jax import lax
from jax.experimental import pallas as pl
from jax.experimental.pallas import tpu as pltpu
from jax.experimental.pallas import tpu_sc as plsc

NUM_ROWS = 16384 * 50
DIM = 32
NUM_CORES = 2
NUM_SUBCORES = 16
NUM_WORKERS = NUM_CORES * NUM_SUBCORES
ROWS_PER_WORKER = NUM_ROWS // NUM_WORKERS
CHUNK = 128
NUM_CHUNKS = ROWS_PER_WORKER // CHUNK
K = 5
GROUP = K * CHUNK
NUM_GROUPS = NUM_CHUNKS // K
NBUF = 4


@functools.partial(
    pl.kernel,
    mesh=plsc.VectorSubcoreMesh(core_axis_name="c", subcore_axis_name="s"),
    out_type=jax.ShapeDtypeStruct((NUM_ROWS, DIM), jnp.float32),
    scratch_types=[
        pltpu.VMEM((ROWS_PER_WORKER,), jnp.int32),
        pltpu.VMEM((NBUF, GROUP, DIM), jnp.float32),
    ]
    + [pltpu.SemaphoreType.DMA] * (2 * NBUF),
    compiler_params=pltpu.CompilerParams(use_tc_tiling_on_sc=False),
)
def _gather_kernel(table_hbm, idx_hbm, out_hbm, idx_v, rows_v, *sems):
    gsem = sems[:NBUF]
    osem = sems[NBUF:]
    wid = lax.axis_index("s") * NUM_CORES + lax.axis_index("c")
    base = wid * ROWS_PER_WORKER
    pltpu.sync_copy(idx_hbm.at[pl.ds(base, ROWS_PER_WORKER)], idx_v)

    def fire(g, b):
        for j in range(K):
            pltpu.async_copy(
                table_hbm.at[idx_v.at[pl.ds((g * K + j) * CHUNK, CHUNK)]],
                rows_v.at[b, pl.ds(j * CHUNK, CHUNK)],
                gsem[b],
            )

    def drain_gathers(g, b):
        for j in range(K):
            pltpu.make_async_copy(
                table_hbm.at[idx_v.at[pl.ds((g * K + j) * CHUNK, CHUNK)]],
                rows_v.at[b, pl.ds(j * CHUNK, CHUNK)],
                gsem[b],
            ).wait()

    def drain_out(b):
        pltpu.make_async_copy(
            rows_v.at[b],
            out_hbm.at[pl.ds(base, GROUP)],
            osem[b],
        ).wait()

    for g in range(NBUF - 1):
        fire(g, g)

    def step(s, b):
        drain_gathers(s, b)
        pltpu.async_copy(
            rows_v.at[b],
            out_hbm.at[pl.ds(base + s * GROUP, GROUP)],
            osem[b],
        )
        bn = (b + NBUF - 1) % NBUF

        @pl.when(s > 0)
        def _():
            drain_out(bn)

        @pl.when(s + NBUF - 1 < NUM_GROUPS)
        def _():
            fire(s + NBUF - 1, bn)

    def body(p, carry):
        for b in range(NBUF):
            step(p * NBUF + b, b)
        return carry

    lax.fori_loop(0, NUM_GROUPS // NBUF, body, 0)
    drain_out((NUM_GROUPS - 1) % NBUF)


def kernel(input, weight):
    idx = input.astype(jnp.int32).reshape(NUM_ROWS)
    out = _gather_kernel(weight, idx)
    return out.reshape(input.shape + (weight.shape[1],))


# speedup vs baseline: 1.6230x; 1.6230x over previous
import functools

import jax
import jax.numpy as jnp
from jax import lax
from jax.experimental import pallas as pl
from jax.experimental.pallas import tpu as pltpu
from jax.experimental.pallas import tpu_sc as plsc

BATCH = 16384
HIST = 50
DIM = 32
NUM_CORES = 2
NUM_SUBCORES = 16
NUM_WORKERS = NUM_CORES * NUM_SUBCORES
B_PER_WORKER = BATCH // NUM_WORKERS
GB = 8
NUM_GROUPS = B_PER_WORKER // GB
NBUF = 4


@functools.partial(
    pl.kernel,
    mesh=plsc.VectorSubcoreMesh(core_axis_name="c", subcore_axis_name="s"),
    out_type=jax.ShapeDtypeStruct((BATCH, HIST, DIM), jnp.float32),
    scratch_types=[
        pltpu.VMEM((B_PER_WORKER, HIST), jnp.int32),
        pltpu.VMEM((NBUF, GB, HIST, DIM), jnp.float32),
    ]
    + [pltpu.SemaphoreType.DMA] * (2 * NBUF),
    compiler_params=pltpu.CompilerParams(use_tc_tiling_on_sc=False),
)
def _gather_kernel(table_hbm, idx_hbm, out_hbm, idx_v, rows_v, *sems):
    gsem = sems[:NBUF]
    osem = sems[NBUF:]
    wid = lax.axis_index("s") * NUM_CORES + lax.axis_index("c")
    base = wid * B_PER_WORKER
    pltpu.sync_copy(idx_hbm.at[pl.ds(base, B_PER_WORKER)], idx_v)

    def fire(g, b):
        for j in range(GB):
            pltpu.async_copy(
                table_hbm.at[idx_v.at[g * GB + j]],
                rows_v.at[b, j],
                gsem[b],
            )

    def drain_gathers(g, b):
        for j in range(GB):
            pltpu.make_async_copy(
                table_hbm.at[idx_v.at[g * GB + j]],
                rows_v.at[b, j],
                gsem[b],
            ).wait()

    def drain_out(b):
        pltpu.make_async_copy(
            rows_v.at[b],
            out_hbm.at[pl.ds(base, GB)],
            osem[b],
        ).wait()

    for g in range(NBUF - 1):
        fire(g, g)

    def step(s, b):
        drain_gathers(s, b)
        pltpu.async_copy(
            rows_v.at[b],
            out_hbm.at[pl.ds(base + s * GB, GB)],
            osem[b],
        )
        bn = (b + NBUF - 1) % NBUF

        @pl.when(s > 0)
        def _():
            drain_out(bn)

        @pl.when(s + NBUF - 1 < NUM_GROUPS)
        def _():
            fire(s + NBUF - 1, bn)

    def body(p, carry):
        for b in range(NBUF):
            step(p * NBUF + b, b)
        return carry

    lax.fori_loop(0, NUM_GROUPS // NBUF, body, 0)
    drain_out((NUM_GROUPS - 1) % NBUF)


def kernel(input, weight):
    return _gather_kernel(weight, input.astype(jnp.int32))
